# baseline (speedup 1.0000x reference)
import functools

import jax
import jax.numpy as jnp
from jax import lax
from jax.experimental import pallas as pl
from jax.experimental.pallas import tpu as pltpu
from jax.experimental.pallas import tpu_sc as plsc

N_OBJS = 100000
D_CODE = 64
D_BCKG = 128
BATCH = 16384

_info = plsc.get_sparse_core_info()
_NC, _NS = _info.num_cores, _info.num_subcores
NW = _NC * _NS
BPW = BATCH // NW
BCHUNK = 32
NBCHUNK = BPW // BCHUNK
ROWS_PER_W = D_CODE * 2 // NW
ICHUNK = 2048
NICHUNK = BATCH // ICHUNK

_mesh = plsc.VectorSubcoreMesh(core_axis_name="c", subcore_axis_name="s")


@functools.partial(
    pl.kernel,
    mesh=_mesh,
    compiler_params=pltpu.CompilerParams(needs_layout_passes=False),
    out_type=[
        jax.ShapeDtypeStruct((D_CODE * BATCH,), jnp.float32),
        jax.ShapeDtypeStruct((D_CODE * BATCH,), jnp.float32),
        jax.ShapeDtypeStruct((BATCH, D_BCKG), jnp.float32),
    ],
    scratch_types=[
        pltpu.VMEM((NBCHUNK, BCHUNK), jnp.int32),
        pltpu.VMEM((2, BCHUNK, D_BCKG), jnp.float32),
        pltpu.VMEM((N_OBJS,), jnp.float32),
        pltpu.VMEM((BATCH,), jnp.int32),
        pltpu.VMEM((2, ICHUNK), jnp.float32),
    ]
    + [pltpu.SemaphoreType.DMA] * 6,
)
def _gather3(ids_hbm, wsT_hbm, waT_hbm, wb_hbm, tmp_s, tmp_a, ob,
             ids_a, bbuf, rowbuf, ids_b, out_b, gs0, gs1, ws0, ws1, as0, as1):
    cid = lax.axis_index("c")
    sid = lax.axis_index("s")
    wid = sid * _NC + cid
    base = wid * BPW
    gsem = (gs0, gs1)
    wsem = (ws0, ws1)
    asem = (as0, as1)

    for j in range(NBCHUNK):
        pltpu.sync_copy(ids_hbm.at[pl.ds(base + j * BCHUNK, BCHUNK)], ids_a.at[j])
    g = [None] * NBCHUNK
    wb = [None] * NBCHUNK

    def _mk_step(j):
        def step():
            s = j % 2
            if j >= 2:
                wb[j - 2].wait()
            g[j] = pltpu.async_copy(wb_hbm.at[ids_a.at[j]], bbuf.at[s], gsem[s])
            if j >= 1:
                g[j - 1].wait()
                wb[j - 1] = pltpu.async_copy(
                    bbuf.at[(j - 1) % 2],
                    ob.at[pl.ds(base + (j - 1) * BCHUNK, BCHUNK)],
                    asem[(j - 1) % 2],
                )
        return step

    def _tail():
        g[NBCHUNK - 1].wait()
        wb[NBCHUNK - 1] = pltpu.async_copy(
            bbuf.at[(NBCHUNK - 1) % 2],
            ob.at[pl.ds(base + (NBCHUNK - 1) * BCHUNK, BCHUNK)],
            asem[(NBCHUNK - 1) % 2],
        )
        wb[NBCHUNK - 2].wait()
        wb[NBCHUNK - 1].wait()

    steps_a = [_mk_step(j) for j in range(NBCHUNK)] + [_tail]

    pltpu.sync_copy(ids_hbm, ids_b)

    def make_gather_block(c, sl):
        def gather_block(i, _):
            start = pl.multiple_of(i * 16, 16)
            idx = ids_b[pl.ds(c * ICHUNK + start, 16)]
            out_b[sl, pl.ds(start, 16)] = plsc.load_gather(rowbuf, [idx])
            return 0
        return gather_block

    k = 0
    for tab, tmp in ((wsT_hbm, tmp_s), (waT_hbm, tmp_a)):
        for p in range(2):
            row = 2 * wid + p
            pltpu.sync_copy(tab.at[row], rowbuf)
            w = [None] * NICHUNK
            for c in range(NICHUNK):
                sl = c % 2
                if c >= 2:
                    w[c - 2].wait()
                lax.fori_loop(0, ICHUNK // 16, make_gather_block(c, sl), 0,
                              unroll=16)
                w[c] = pltpu.async_copy(
                    out_b.at[sl],
                    tmp.at[pl.ds(row * BATCH + c * ICHUNK, ICHUNK)],
                    wsem[sl],
                )
                if k < len(steps_a):
                    steps_a[k]()
                    k += 1
            w[NICHUNK - 2].wait()
            w[NICHUNK - 1].wait()
    for step in steps_a[k:]:
        step()


def kernel(instance_ids, W_shape, W_app, W_bckg):
    ids = jnp.squeeze(instance_ids).astype(jnp.int32)
    tmp_s, tmp_a, ob = _gather3(ids, W_shape.T, W_app.T, W_bckg)
    emb_shape = tmp_s.reshape(D_CODE, BATCH).T
    emb_app = tmp_a.reshape(D_CODE, BATCH).T
    return (emb_shape, emb_app, ob)


# speedup vs baseline: 1.0164x; 1.0164x over previous
import functools

import jax
import jax.numpy as jnp
from jax import lax
from jax.experimental import pallas as pl
from jax.experimental.pallas import tpu as pltpu
from jax.experimental.pallas import tpu_sc as plsc

N_OBJS = 100000
D_CODE = 64
D_BCKG = 128
BATCH = 16384

_info = plsc.get_sparse_core_info()
_NC, _NS = _info.num_cores, _info.num_subcores
NW = _NC * _NS
BPW = BATCH // NW
BCHUNK = 32
NBCHUNK = BPW // BCHUNK
ROWS_PER_W = D_CODE * 2 // NW
ICHUNK = 2048
NICHUNK = BATCH // ICHUNK

_mesh = plsc.VectorSubcoreMesh(core_axis_name="c", subcore_axis_name="s")


@functools.partial(
    pl.kernel,
    mesh=_mesh,
    compiler_params=pltpu.CompilerParams(needs_layout_passes=False),
    out_type=[
        jax.ShapeDtypeStruct((D_CODE * BATCH,), jnp.float32),
        jax.ShapeDtypeStruct((D_CODE * BATCH,), jnp.float32),
        jax.ShapeDtypeStruct((BATCH, D_BCKG), jnp.float32),
    ],
    scratch_types=[
        pltpu.VMEM((NBCHUNK, BCHUNK), jnp.int32),
        pltpu.VMEM((2, BCHUNK, D_BCKG), jnp.float32),
        pltpu.VMEM((N_OBJS,), jnp.float32),
        pltpu.VMEM((BATCH,), jnp.int32),
        pltpu.VMEM((2, ICHUNK), jnp.float32),
    ]
    + [pltpu.SemaphoreType.DMA] * 6,
)
def _gather3(ids_hbm, wsT_hbm, waT_hbm, wb_hbm, tmp_s, tmp_a, ob,
             ids_a, bbuf, rowbuf, ids_b, out_b, gs0, gs1, ws0, ws1, as0, as1):
    cid = lax.axis_index("c")
    sid = lax.axis_index("s")
    wid = sid * _NC + cid
    base = wid * BPW
    gsem = (gs0, gs1)
    wsem = (ws0, ws1)
    asem = (as0, as1)

    for j in range(NBCHUNK):
        pltpu.sync_copy(ids_hbm.at[pl.ds(base + j * BCHUNK, BCHUNK)], ids_a.at[j])
    g = [None] * NBCHUNK
    wb = [None] * NBCHUNK

    def _mk_step(j):
        def step():
            s = j % 2
            if j >= 2:
                wb[j - 2].wait()
            g[j] = pltpu.async_copy(wb_hbm.at[ids_a.at[j]], bbuf.at[s], gsem[s])
            if j >= 1:
                g[j - 1].wait()
                wb[j - 1] = pltpu.async_copy(
                    bbuf.at[(j - 1) % 2],
                    ob.at[pl.ds(base + (j - 1) * BCHUNK, BCHUNK)],
                    asem[(j - 1) % 2],
                )
        return step

    def _tail():
        g[NBCHUNK - 1].wait()
        wb[NBCHUNK - 1] = pltpu.async_copy(
            bbuf.at[(NBCHUNK - 1) % 2],
            ob.at[pl.ds(base + (NBCHUNK - 1) * BCHUNK, BCHUNK)],
            asem[(NBCHUNK - 1) % 2],
        )
        wb[NBCHUNK - 2].wait()
        wb[NBCHUNK - 1].wait()

    steps_a = [_mk_step(j) for j in range(NBCHUNK)] + [_tail]

    pltpu.sync_copy(ids_hbm, ids_b)

    def make_gather_block(c, sl):
        def gather_block(i, _):
            start = pl.multiple_of(i * 16, 16)
            idx = ids_b[pl.ds(c * ICHUNK + start, 16)]
            out_b[sl, pl.ds(start, 16)] = plsc.load_gather(rowbuf, [idx])
            return 0
        return gather_block

    k = 0
    for tab, tmp in ((wsT_hbm, tmp_s), (waT_hbm, tmp_a)):
        for p in range(2):
            row = 2 * wid + p
            pltpu.sync_copy(tab.at[row], rowbuf)
            w = [None] * NICHUNK
            for c in range(NICHUNK):
                sl = c % 2
                if c >= 2:
                    w[c - 2].wait()
                lax.fori_loop(0, ICHUNK // 16, make_gather_block(c, sl), 0,
                              unroll=8)
                w[c] = pltpu.async_copy(
                    out_b.at[sl],
                    tmp.at[pl.ds(row * BATCH + c * ICHUNK, ICHUNK)],
                    wsem[sl],
                )
                if k < len(steps_a):
                    steps_a[k]()
                    k += 1
            w[NICHUNK - 2].wait()
            w[NICHUNK - 1].wait()
    for step in steps_a[k:]:
        step()


def kernel(instance_ids, W_shape, W_app, W_bckg):
    ids = jnp.squeeze(instance_ids).astype(jnp.int32)
    tmp_s, tmp_a, ob = _gather3(ids, W_shape.T, W_app.T, W_bckg)
    emb_shape = tmp_s.reshape(D_CODE, BATCH).T
    emb_app = tmp_a.reshape(D_CODE, BATCH).T
    return (emb_shape, emb_app, ob)
